# baseline (speedup 1.0000x reference)
import functools

import jax
import jax.numpy as jnp
from jax import lax
from jax.experimental import pallas as pl
from jax.experimental.pallas import tpu as pltpu
from jax.experimental.pallas import tpu_sc as plsc

ALPHA = 0.2
LANES = 16
NCORES = 2
NSUB = 16
NW = NCORES * NSUB
GRP = 64
UNROLL = 8
NBUF = 2

_SC_PARAMS = pltpu.CompilerParams(needs_layout_passes=False)


def _rs_rows(n, d):
    return -(-(n + 1) // d)


def _tc_prep(x, W, aT, *, bm):
    n, d = x.shape

    def body(x_ref, w_ref, at_ref, h_ref, svec_ref):
        h = jnp.dot(x_ref[...], w_ref[...],
                    preferred_element_type=jnp.float32,
                    precision=lax.Precision.HIGHEST)
        sv = jnp.dot(h, at_ref[...],
                     preferred_element_type=jnp.float32,
                     precision=lax.Precision.HIGHEST)
        h_ref[...] = h
        svec_ref[...] = sv

    return pl.pallas_call(
        body,
        grid=(n // bm,),
        in_specs=[
            pl.BlockSpec((bm, d), lambda i: (i, 0)),
            pl.BlockSpec((d, d), lambda i: (0, 0)),
            pl.BlockSpec((d, 2), lambda i: (0, 0)),
        ],
        out_specs=[
            pl.BlockSpec((bm, d), lambda i: (i, 0)),
            pl.BlockSpec((bm, 2), lambda i: (i, 0)),
        ],
        out_shape=[
            jax.ShapeDtypeStruct((n, d), jnp.float32),
            jax.ShapeDtypeStruct((n, 2), jnp.float32),
        ],
    )(x, W, aT)


def _tc_combine(acc, rs, *, bm, d):
    _, n, _ = acc.shape

    def body(acc_ref, rs_ref, out_ref):
        hp = acc_ref[0] + acc_ref[1]
        rsum = jnp.sum(rs_ref[...], axis=1)[:, None] + 1e-5
        v = hp / rsum
        out_ref[...] = jnp.where(v > 0, v, jnp.exp(v) - 1.0)

    return pl.pallas_call(
        body,
        grid=(n // bm,),
        in_specs=[
            pl.BlockSpec((2, bm, d), lambda i: (0, i, 0)),
            pl.BlockSpec((bm, NW), lambda i: (i, 0)),
        ],
        out_specs=pl.BlockSpec((bm, d), lambda i: (i, 0)),
        out_shape=jax.ShapeDtypeStruct((n, d), jnp.float32),
    )(acc, rs)


def _sc_edge_accum(h, hs, hd, src, dst, *, n, d, ng):
    mesh = plsc.VectorSubcoreMesh(
        core_axis_name="c", subcore_axis_name="s",
        num_cores=NCORES, num_subcores=NSUB)
    nsteps = ng // UNROLL
    npad = n + 8
    chunk = 400
    nchunk = n // chunk
    nsb = GRP // LANES
    blk = UNROLL * GRP

    @functools.partial(
        pl.kernel,
        out_type=[
            jax.ShapeDtypeStruct((NCORES, n, d), jnp.float32),
            jax.ShapeDtypeStruct((NW, _rs_rows(n, d), d), jnp.float32),
        ],
        mesh=mesh,
        scratch_types=[
            pltpu.VMEM((n + LANES,), jnp.float32),
            pltpu.VMEM((n + LANES,), jnp.float32),
            pltpu.VMEM((_rs_rows(n, d), d), jnp.float32),
            pltpu.VMEM((blk,), jnp.int32),
            pltpu.VMEM((blk,), jnp.int32),
            [pltpu.VMEM((GRP, d), jnp.float32) for _ in range(NBUF)],
            pltpu.VMEM_SHARED((npad, d), jnp.float32),
            [pltpu.SemaphoreType.DMA for _ in range(NBUF)],
            [pltpu.SemaphoreType.DMA for _ in range(NBUF)],
        ],
        compiler_params=_SC_PARAMS,
    )
    def k(h_h, hs_h, hd_h, src_h, dst_h, out_h, rs_h,
          hs_v, hd_v, rs_v, src_v, dst_v, rows, acc, gsem, ssem):
        c = lax.axis_index("c")
        s = lax.axis_index("s")
        wid = s * NCORES + c

        pltpu.sync_copy(hs_h, hs_v.at[pl.ds(0, n)])
        pltpu.sync_copy(hd_h, hd_v.at[pl.ds(0, n)])
        z = jnp.zeros((LANES,), jnp.float32)
        hs_v[pl.ds(n, LANES)] = z
        hd_v[pl.ds(n, LANES)] = z

        def zrs(i, carry):
            for f in range(d // LANES):
                rs_v[i, pl.ds(f * LANES, LANES)] = z
            return carry

        lax.fori_loop(0, _rs_rows(n, d), zrs, 0)

        def zrow(r, carry):
            for f in range(d // LANES):
                rows[0][r, pl.ds(f * LANES, LANES)] = z
            return carry

        lax.fori_loop(0, GRP, zrow, 0)
        for ch in range(nchunk):
            @pl.when(s == ch % NSUB)
            def _():
                base = ch * chunk
                nfull = chunk // GRP
                for r6 in range(nfull):
                    pltpu.sync_copy(rows[0],
                                    acc.at[pl.ds(base + r6 * GRP, GRP)])
                rem = chunk - nfull * GRP
                if rem:
                    pltpu.sync_copy(
                        rows[0].at[pl.ds(0, rem)],
                        acc.at[pl.ds(base + nfull * GRP, rem)])
        plsc.subcore_barrier()

        def compute_group(j):
            buf = rows[j % NBUF]

            def sub(sb, carry2):
                eoff = pl.ds(
                    pl.multiple_of(j * GRP + sb * LANES, LANES), LANES)
                src16 = src_v[eoff]
                dst16 = dst_v[eoff]
                lg = (plsc.load_gather(hs_v, [src16])
                      + plsc.load_gather(hd_v, [dst16]))
                e16 = jnp.exp(-jnp.maximum(lg, ALPHA * lg))
                plsc.addupdate_scatter(
                    rs_v,
                    [lax.shift_right_logical(src16, d.bit_length() - 1),
                     lax.bitwise_and(src16, jnp.int32(d - 1))],
                    e16)
                rbase = pl.multiple_of(sb * LANES, LANES)
                for jj in range(LANES):
                    scale = jnp.broadcast_to(e16[jj], (LANES,))
                    r = rbase + jj
                    for f in range(d // LANES):
                        sl = pl.ds(f * LANES, LANES)
                        buf[r, sl] = buf[r, sl] * scale
                pltpu.async_copy(
                    buf.at[pl.ds(rbase, LANES)], acc.at[src16],
                    ssem[j % NBUF], add=True)
                return carry2

            lax.fori_loop(0, nsb, sub, 0)

        def gath(j):
            off = pl.ds(pl.multiple_of(j * GRP, GRP), GRP)
            pltpu.async_copy(h_h.at[dst_v.at[off]], rows[j % NBUF],
                             gsem[j % NBUF])

        def wait_gath(j):
            pltpu.make_async_copy(
                h_h.at[pl.ds(0, GRP)], rows[j % NBUF], gsem[j % NBUF]).wait()

        def wait_scat(j):
            pltpu.make_async_copy(
                h_h.at[pl.ds(0, GRP)], rows[j % NBUF], ssem[j % NBUF]).wait()

        def step(kk, carry):
            pltpu.sync_copy(src_h.at[wid, kk], src_v)
            pltpu.sync_copy(dst_h.at[wid, kk], dst_v)
            first = kk == 0

            @pl.when(jnp.logical_not(first))
            def _():
                wait_scat(0)
            gath(0)
            for j in range(UNROLL):
                if j + 1 < UNROLL:
                    if j == 0:
                        @pl.when(jnp.logical_not(first))
                        def _():
                            wait_scat(1)
                    else:
                        wait_scat(j + 1)
                    gath(j + 1)
                wait_gath(j)
                compute_group(j)
            return carry

        lax.fori_loop(0, nsteps, step, 0)
        for j in range(NBUF):
            wait_scat(j)

        pltpu.sync_copy(rs_v, rs_h.at[wid])

        plsc.subcore_barrier()
        for ch in range(nchunk):
            @pl.when(s == ch % NSUB)
            def _():
                pltpu.sync_copy(acc.at[pl.ds(ch * chunk, chunk)],
                                out_h.at[c, pl.ds(ch * chunk, chunk)])

    return k(h, hs, hd, src, dst)


@jax.jit
def kernel(x, edge_index, W, a):
    n, d = x.shape
    e = edge_index.shape[1]
    epw = e // NW
    pad = (-epw) % (GRP * UNROLL)
    epw_pad = epw + pad
    ng = epw_pad // GRP
    nsteps = ng // UNROLL

    aT = jnp.stack([a[0, :d], a[0, d:]], axis=1)
    h, svec = _tc_prep(x, W, aT, bm=400)
    srcr = jnp.concatenate(
        [edge_index[0].reshape(NW, epw),
         jnp.full((NW, pad), n, jnp.int32)], axis=1)
    dstr = jnp.concatenate(
        [edge_index[1].reshape(NW, epw),
         jnp.zeros((NW, pad), jnp.int32)], axis=1)
    acc, rs = _sc_edge_accum(
        h, svec[:, 0], svec[:, 1],
        srcr.reshape(NW, nsteps, UNROLL * GRP),
        dstr.reshape(NW, nsteps, UNROLL * GRP),
        n=n, d=d, ng=ng)
    rs_t = rs.reshape(NW, -1)[:, :n].T
    return _tc_combine(acc, rs_t, bm=400, d=d)


# speedup vs baseline: 1.3303x; 1.3303x over previous
import functools

import jax
import jax.numpy as jnp
from jax import lax
from jax.experimental import pallas as pl
from jax.experimental.pallas import tpu as pltpu
from jax.experimental.pallas import tpu_sc as plsc

ALPHA = 0.2
LANES = 16
NCORES = 2
NSUB = 16
NW = NCORES * NSUB
GRP = 80
UNROLL = 6
ACHUNK = 1008

_SC_PARAMS = pltpu.CompilerParams(
    needs_layout_passes=False, use_tc_tiling_on_sc=False)


def _tc_prep(x, W, aT, *, bm, dp):
    n, d = x.shape

    def body(x_ref, w_ref, at_ref, hext_ref, svec_ref):
        h = jnp.dot(x_ref[...], w_ref[...],
                    preferred_element_type=jnp.float32,
                    precision=lax.Precision.HIGHEST)
        sv = jnp.dot(h, at_ref[...],
                     preferred_element_type=jnp.float32,
                     precision=lax.Precision.HIGHEST)
        ones = jnp.ones((bm, 1), jnp.float32)
        zpad = jnp.zeros((bm, dp - d - 1), jnp.float32)
        hext_ref[...] = jnp.concatenate([h, ones, zpad], axis=1)
        svec_ref[...] = sv

    return pl.pallas_call(
        body,
        grid=(n // bm,),
        in_specs=[
            pl.BlockSpec((bm, d), lambda i: (i, 0)),
            pl.BlockSpec((d, d), lambda i: (0, 0)),
            pl.BlockSpec((d, 2), lambda i: (0, 0)),
        ],
        out_specs=[
            pl.BlockSpec((bm, dp), lambda i: (i, 0)),
            pl.BlockSpec((bm, 2), lambda i: (i, 0)),
        ],
        out_shape=[
            jax.ShapeDtypeStruct((n, dp), jnp.float32),
            jax.ShapeDtypeStruct((n, 2), jnp.float32),
        ],
    )(x, W, aT)


def _tc_combine(acc, *, bm, d):
    _, n, dp = acc.shape

    def body(acc_ref, out_ref):
        a0 = acc_ref[0]
        a1 = acc_ref[1]
        hp = a0[:, :d] + a1[:, :d]
        rs = a0[:, d:d + 1] + a1[:, d:d + 1] + 1e-5
        v = hp / rs
        out_ref[...] = jnp.where(v > 0, v, jnp.exp(v) - 1.0)

    return pl.pallas_call(
        body,
        grid=(n // bm,),
        in_specs=[pl.BlockSpec((2, bm, dp), lambda i: (0, i, 0))],
        out_specs=pl.BlockSpec((bm, d), lambda i: (i, 0)),
        out_shape=jax.ShapeDtypeStruct((n, d), jnp.float32),
    )(acc)


def _sc_edge_coeff(hs, hd, sd, *, n, epw):
    mesh = plsc.VectorSubcoreMesh(
        core_axis_name="c", subcore_axis_name="s",
        num_cores=NCORES, num_subcores=NSUB)
    nchunks = epw // ACHUNK

    @functools.partial(
        pl.kernel,
        out_type=jax.ShapeDtypeStruct((NW, epw), jnp.float32),
        mesh=mesh,
        scratch_types=[
            pltpu.VMEM((n + LANES,), jnp.float32),
            pltpu.VMEM((n + LANES,), jnp.float32),
            pltpu.VMEM((ACHUNK,), jnp.int32),
            pltpu.VMEM((ACHUNK,), jnp.int32),
            pltpu.VMEM((ACHUNK,), jnp.float32),
        ],
        compiler_params=_SC_PARAMS,
    )
    def k(hs_h, hd_h, sd_h, e_h, hs_v, hd_v, s_v, d_v, e_v):
        c = lax.axis_index("c")
        s = lax.axis_index("s")
        wid = s * NCORES + c
        pltpu.sync_copy(hs_h, hs_v.at[pl.ds(0, n)])
        pltpu.sync_copy(hd_h, hd_v.at[pl.ds(0, n)])
        z16 = jnp.zeros((LANES,), jnp.float32)
        hs_v[pl.ds(n, LANES)] = z16
        hd_v[pl.ds(n, LANES)] = z16

        def chunk(ci, carry):
            base = pl.multiple_of(ci * ACHUNK, ACHUNK)
            pltpu.sync_copy(sd_h.at[wid, 0, pl.ds(base, ACHUNK)], s_v)
            pltpu.sync_copy(sd_h.at[wid, 1, pl.ds(base, ACHUNK)], d_v)

            def blk(b, carry2):
                off = pl.ds(pl.multiple_of(b * LANES, LANES), LANES)
                lg = (plsc.load_gather(hs_v, [s_v[off]])
                      + plsc.load_gather(hd_v, [d_v[off]]))
                e_v[off] = jnp.exp(-jnp.maximum(lg, ALPHA * lg))
                return carry2

            lax.fori_loop(0, ACHUNK // LANES, blk, 0)
            pltpu.sync_copy(e_v, e_h.at[wid, pl.ds(base, ACHUNK)])
            return carry

        lax.fori_loop(0, nchunks, chunk, 0)

    return k(hs, hd, sd)


def _sc_edge_accum(hext, sd, ev, *, n, dp, ng):
    mesh = plsc.VectorSubcoreMesh(
        core_axis_name="c", subcore_axis_name="s",
        num_cores=NCORES, num_subcores=NSUB)
    nsteps = ng // UNROLL
    npad = n + 8
    chunk = 400
    nchunk = n // chunk
    nsb = GRP // LANES

    @functools.partial(
        pl.kernel,
        out_type=jax.ShapeDtypeStruct((NCORES, n, dp), jnp.float32),
        mesh=mesh,
        scratch_types=[
            pltpu.VMEM((UNROLL, 2, GRP), jnp.int32),
            pltpu.VMEM((UNROLL, GRP), jnp.float32),
            [pltpu.VMEM((GRP, dp), jnp.float32) for _ in range(3)],
            pltpu.VMEM_SHARED((npad, dp), jnp.float32),
            [pltpu.SemaphoreType.DMA for _ in range(3)],
            [pltpu.SemaphoreType.DMA for _ in range(3)],
        ],
        compiler_params=_SC_PARAMS,
    )
    def k(hext_h, sd_h, e_h, out_h, sd_v, e_v, rows, acc, gsem, ssem):
        c = lax.axis_index("c")
        s = lax.axis_index("s")
        wid = s * NCORES + c

        z = jnp.zeros((LANES,), jnp.float32)

        def zrow(r, carry):
            for f in range((dp - 8) // LANES):
                rows[0][r, pl.ds(f * LANES, LANES)] = z
            rows[0][r, pl.ds(dp - LANES, LANES)] = z
            return carry

        lax.fori_loop(0, GRP, zrow, 0)
        for ch in range(nchunk):
            @pl.when(s == ch % NSUB)
            def _():
                for r5 in range(chunk // GRP):
                    pltpu.sync_copy(
                        rows[0],
                        acc.at[pl.ds(ch * chunk + r5 * GRP, GRP)])
        plsc.subcore_barrier()

        def compute_group(j):
            buf = rows[j % 3]
            lanes_lo = lax.iota(jnp.int32, LANES) < 8

            def sub(sb, carry2):
                sbo = pl.ds(pl.multiple_of(sb * LANES, LANES), LANES)
                src16 = sd_v[j, 0, sbo]
                e16 = e_v[j, sbo]
                rbase = pl.multiple_of(sb * LANES, LANES)
                for jj in range(LANES):
                    scale = jnp.broadcast_to(e16[jj], (LANES,))
                    r = rbase + jj
                    for f in range((dp - 8) // LANES):
                        sl = pl.ds(f * LANES, LANES)
                        buf[r, sl] = buf[r, sl] * scale
                    tl = pl.ds(dp - LANES, LANES)
                    v = buf[r, tl]
                    buf[r, tl] = jnp.where(lanes_lo, v, v * scale)
                pltpu.async_copy(
                    buf.at[sbo], acc.at[src16], ssem[j % 3], add=True)
                return carry2

            lax.fori_loop(0, nsb, sub, 0)

        def gath(j):
            pltpu.async_copy(
                hext_h.at[sd_v.at[j, 1]], rows[j % 3], gsem[j % 3])

        def wait_gath(j):
            pltpu.make_async_copy(
                hext_h.at[pl.ds(0, GRP)], rows[j % 3], gsem[j % 3]).wait()

        def wait_scat(j):
            pltpu.make_async_copy(
                hext_h.at[pl.ds(0, GRP)], rows[j % 3], ssem[j % 3]).wait()

        def step(kk, carry):
            pltpu.sync_copy(sd_h.at[wid, kk], sd_v)
            pltpu.sync_copy(e_h.at[wid, kk], e_v)
            first = kk == 0

            for j in range(3):
                @pl.when(jnp.logical_not(first))
                def _():
                    wait_scat(j)
                gath(j)
            for j in range(3):
                wait_gath(j)
                compute_group(j)
                wait_scat(j)
                gath(j + 3)
            for j in range(3, UNROLL):
                wait_gath(j)
                compute_group(j)
            return carry

        lax.fori_loop(0, nsteps, step, 0)
        for j in range(3):
            wait_scat(j)

        plsc.subcore_barrier()
        for ch in range(nchunk):
            @pl.when(s == ch % NSUB)
            def _():
                pltpu.sync_copy(acc.at[pl.ds(ch * chunk, chunk)],
                                out_h.at[c, pl.ds(ch * chunk, chunk)])

    return k(hext, sd, ev)


@jax.jit
def kernel(x, edge_index, W, a):
    n, d = x.shape
    e = edge_index.shape[1]
    dp = d + 8
    epw = e // NW
    epw_pad = epw + GRP
    ng = epw_pad // GRP
    nsteps = ng // UNROLL

    aT = jnp.stack([a[0, :d], a[0, d:]], axis=1)
    hext, svec = _tc_prep(x, W, aT, bm=400, dp=dp)
    srcr = jnp.concatenate(
        [edge_index[0].reshape(NW, epw),
         jnp.full((NW, GRP), n, jnp.int32)], axis=1)
    dstr = jnp.concatenate(
        [edge_index[1].reshape(NW, epw),
         jnp.zeros((NW, GRP), jnp.int32)], axis=1)
    sd = jnp.stack([srcr, dstr], axis=1)
    ev = _sc_edge_coeff(svec[:, 0], svec[:, 1], sd, n=n, epw=epw_pad)
    sdB = jnp.stack([srcr.reshape(NW, nsteps, UNROLL, GRP),
                     dstr.reshape(NW, nsteps, UNROLL, GRP)],
                    axis=3)
    acc = _sc_edge_accum(hext, sdB, ev.reshape(NW, nsteps, UNROLL, GRP),
                         n=n, dp=dp, ng=ng)
    return _tc_combine(acc, bm=400, d=d)
